# Initial kernel scaffold; baseline (speedup 1.0000x reference)
import jax
import jax.numpy as jnp
from jax.experimental import pallas as pl


def kernel(features, edge_index, W1, W2, Wfc):
    raise NotImplementedError("write your pallas kernel here")


# speedup vs baseline: 6.5106x; 6.5106x over previous
import functools

import jax
import jax.numpy as jnp
from jax import lax
from jax.experimental import pallas as pl
from jax.experimental.pallas import tpu as pltpu
from jax.experimental.pallas import tpu_sc as plsc

N_NODES = 10000
D = 128
OUT = 8

NC, NS = 2, 16
NTILES = NC * NS
CHUNK = 128
CHUNKS_PER_TILE = 79
EPT = CHUNK * CHUNKS_PER_TILE
E_PAD = EPT * NTILES
PAD_ROW = N_NODES
NPAD = 10240
ROWS_PER_TILE = 624
TAIL = N_NODES - NS * ROWS_PER_TILE
TAIL_BASE = NS * ROWS_PER_TILE
DROWS_PER_TILE = NPAD // NS
DEG_W = 16

_AGG_PIECES = ((0, 128), (128, 128), (256, 128), (384, 128), (512, 112))

_MESH = plsc.VectorSubcoreMesh(core_axis_name="c", subcore_axis_name="s")


@functools.partial(
    pl.kernel,
    out_type=jax.ShapeDtypeStruct((NC, N_NODES, D), jnp.float32),
    mesh=_MESH,
    scratch_types=(
        pltpu.VMEM((CHUNK,), jnp.int32),
        pltpu.VMEM((CHUNK,), jnp.int32),
        pltpu.VMEM((CHUNK, D), jnp.float32),
        pltpu.VMEM_SHARED((N_NODES, D), jnp.float32),
        pltpu.SemaphoreType.DMA,
    ),
)
def _sc_agg(feat, srcp, dstp,
            agg_out,
            sidx, didx, rows, agg_sh, sem):
    c = lax.axis_index("c")
    s = lax.axis_index("s")
    wid = s * NC + c
    base = s * ROWS_PER_TILE

    zero16 = jnp.zeros((16,), jnp.float32)

    def fill(r, carry):
        for cc in range(D // 16):
            rows[r, pl.ds(cc * 16, 16)] = zero16
        return carry

    lax.fori_loop(0, CHUNK, fill, 0)

    for j, sz in _AGG_PIECES:
        pltpu.sync_copy(rows.at[pl.ds(0, sz)], agg_sh.at[pl.ds(base + j, sz)])

    @pl.when(s == NS - 1)
    def _():
        pltpu.sync_copy(rows.at[pl.ds(0, TAIL)],
                        agg_sh.at[pl.ds(TAIL_BASE, TAIL)])

    plsc.subcore_barrier()

    def body(ci, carry):
        off = wid * EPT + ci * CHUNK
        pltpu.sync_copy(srcp.at[pl.ds(off, CHUNK)], sidx)
        pltpu.sync_copy(dstp.at[pl.ds(off, CHUNK)], didx)
        pltpu.async_copy(feat.at[sidx], rows, sem).wait()
        pltpu.sync_copy(rows, agg_sh.at[didx], add=True)
        return carry

    lax.fori_loop(0, CHUNKS_PER_TILE, body, 0)
    plsc.subcore_barrier()

    for j, sz in _AGG_PIECES:
        pltpu.sync_copy(agg_sh.at[pl.ds(base + j, sz)], rows.at[pl.ds(0, sz)])
        pltpu.sync_copy(rows.at[pl.ds(0, sz)],
                        agg_out.at[c, pl.ds(base + j, sz)])

    @pl.when(s == NS - 1)
    def _():
        pltpu.sync_copy(agg_sh.at[pl.ds(TAIL_BASE, TAIL)],
                        rows.at[pl.ds(0, TAIL)])
        pltpu.sync_copy(rows.at[pl.ds(0, TAIL)],
                        agg_out.at[c, pl.ds(TAIL_BASE, TAIL)])


@functools.partial(
    pl.kernel,
    out_type=jax.ShapeDtypeStruct((NC, NPAD, D), jnp.float32),
    mesh=_MESH,
    scratch_types=(
        pltpu.VMEM((CHUNK,), jnp.int32),
        pltpu.VMEM((CHUNK, D), jnp.float32),
        pltpu.VMEM_SHARED((NPAD, D), jnp.float32),
    ),
)
def _sc_deg(srcp, deg_out, sidx, ones_v, deg_sh):
    c = lax.axis_index("c")
    s = lax.axis_index("s")
    wid = s * NC + c
    dbase = s * DROWS_PER_TILE

    zero16 = jnp.zeros((16,), jnp.float32)

    def fillz(r, carry):
        for cc in range(D // 16):
            ones_v[r, pl.ds(cc * 16, 16)] = zero16
        return carry

    lax.fori_loop(0, CHUNK, fillz, 0)
    for j in range(DROWS_PER_TILE // CHUNK):
        pltpu.sync_copy(ones_v, deg_sh.at[pl.ds(dbase + j * CHUNK, CHUNK)])

    one16 = jnp.ones((16,), jnp.float32)

    def fill1(r, carry):
        for cc in range(D // 16):
            ones_v[r, pl.ds(cc * 16, 16)] = one16
        return carry

    lax.fori_loop(0, CHUNK, fill1, 0)
    plsc.subcore_barrier()

    def body(ci, carry):
        off = wid * EPT + ci * CHUNK
        pltpu.sync_copy(srcp.at[pl.ds(off, CHUNK)], sidx)
        pltpu.sync_copy(ones_v, deg_sh.at[sidx], add=True)
        return carry

    lax.fori_loop(0, CHUNKS_PER_TILE, body, 0)
    plsc.subcore_barrier()

    for j in range(DROWS_PER_TILE // CHUNK):
        pltpu.sync_copy(deg_sh.at[pl.ds(dbase + j * CHUNK, CHUNK)], ones_v)
        pltpu.sync_copy(ones_v, deg_out.at[c, pl.ds(dbase + j * CHUNK, CHUNK)])


BLK = 1000
GRID = N_NODES // BLK


def _tc_body(aggp_ref, deg_ref, W1_ref, W2_ref, Wfc_ref, out_ref, gacc):
    i = pl.program_id(0)

    @pl.when(i == 0)
    def _():
        gacc[...] = jnp.zeros_like(gacc)

    a = aggp_ref[0] + aggp_ref[1]
    h = jnp.maximum(
        jnp.dot(a, W1_ref[...], preferred_element_type=jnp.float32,
                precision=lax.Precision.HIGHEST), 0.0)
    w3 = deg_ref[0, :, 0:1] + deg_ref[1, :, 0:1]
    gacc[...] += jnp.sum(w3 * h, axis=0, keepdims=True)

    @pl.when(i == GRID - 1)
    def _():
        g = gacc[...] * (1.0 / N_NODES)
        z = jnp.dot(g, W2_ref[...], preferred_element_type=jnp.float32,
                    precision=lax.Precision.HIGHEST)
        o = jnp.dot(z, Wfc_ref[...], preferred_element_type=jnp.float32,
                    precision=lax.Precision.HIGHEST)
        out_ref[...] = jax.nn.sigmoid(o)


_tc_final = pl.pallas_call(
    _tc_body,
    grid=(GRID,),
    in_specs=[
        pl.BlockSpec((NC, BLK, D), lambda i: (0, i, 0)),
        pl.BlockSpec((NC, BLK, D), lambda i: (0, i, 0)),
        pl.BlockSpec((D, D), lambda i: (0, 0)),
        pl.BlockSpec((D, D), lambda i: (0, 0)),
        pl.BlockSpec((D, OUT), lambda i: (0, 0)),
    ],
    out_specs=pl.BlockSpec((1, OUT), lambda i: (0, 0)),
    out_shape=jax.ShapeDtypeStruct((1, OUT), jnp.float32),
    scratch_shapes=[pltpu.VMEM((1, D), jnp.float32)],
)


def kernel(features, edge_index, W1, W2, Wfc):
    src = edge_index[0].astype(jnp.int32)
    dst = edge_index[1].astype(jnp.int32)
    e = src.shape[0]
    feat_pad = jnp.concatenate(
        [features, jnp.zeros((NPAD - N_NODES, D), jnp.float32)], axis=0)
    srcp = jnp.concatenate([src, jnp.full((E_PAD - e,), PAD_ROW, jnp.int32)])
    dstp = jnp.concatenate([dst, jnp.zeros((E_PAD - e,), jnp.int32)])
    aggp = _sc_agg(feat_pad, srcp, dstp)
    degp = _sc_deg(srcp)
    return _tc_final(aggp, degp[:, :N_NODES, :], W1, W2, Wfc)
